# baseline (speedup 1.0000x reference)
import functools

import numpy as np
import jax
import jax.numpy as jnp
from jax import lax
from jax.experimental import pallas as pl
from jax.experimental.pallas import tpu as pltpu
from jax.experimental.pallas import tpu_sc as plsc

_DIM = 256; _DFF = 2048; _NL = 4; _NH = 8; _NP = 4; _DH = 32
_B = 4; _LQ = 900; _R = _B * _LQ
_SHAPES = np.array([[128, 128], [64, 64], [32, 32], [16, 16]], np.int64)
_LIN = int((_SHAPES[:, 0] * _SHAPES[:, 1]).sum())
_LSTART = np.concatenate([[0], np.cumsum(_SHAPES[:, 0] * _SHAPES[:, 1])[:-1]]).astype(np.int64)
_NROWS = _B * _LIN * _NH
_LP = _NL * _NP

_l_lane = np.tile(np.repeat(np.arange(_NL), _NP), _NH)
_h_lane = np.repeat(np.arange(_NH), _LP)
_Wi_c = _SHAPES[_l_lane, 1].astype(np.int32)[None, :]
_Hi_c = _SHAPES[_l_lane, 0].astype(np.int32)[None, :]
_base_c = (_LSTART[_l_lane] * _NH + _h_lane).astype(np.int32)[None, :]
_gsum_c = np.zeros((128, 8), np.float32)
_gsum_c[np.arange(128), _h_lane] = 1.0
_gbc_c = _gsum_c.T.copy()
_mx_c = np.zeros((8, 128), np.float32)
_mx_c[_l_lane * 2, np.arange(128)] = 1.0
_my_c = np.zeros((8, 128), np.float32)
_my_c[_l_lane * 2 + 1, np.arange(128)] = 1.0
_ex_c = np.zeros((128, 256), np.float32)
_ex_c[np.arange(128), 2 * np.arange(128)] = 1.0
_ey_c = np.zeros((128, 256), np.float32)
_ey_c[np.arange(128), 2 * np.arange(128) + 1] = 1.0
_sx_c = np.zeros((256, 128), np.float32)
_sx_c[2 * np.arange(128), np.arange(128)] = 1.0
_sy_c = np.zeros((256, 128), np.float32)
_sy_c[2 * np.arange(128) + 1, np.arange(128)] = 1.0
_collo_c = np.empty((128,), np.int32)
_colhi_c = np.empty((128,), np.int32)
for _h in range(_NH):
    for _k in range(16):
        _collo_c[_h * 16 + _k] = _h * 32 + _k
        _colhi_c[_h * 16 + _k] = _h * 32 + 16 + _k

_NAW = 30; _CR = 16; _GC = 6; _NGRP = 5; _NCH = _GC * _NGRP
_BS = 128


def _prep_body(x_ref, pos_ref, ref8_ref, bcol_ref, g1_ref, bb1_ref,
               woff_ref, boff_ref, wa_ref, ba_ref,
               wi_ref, hi_ref, base_ref, gsum_ref, gbc_ref,
               mx_ref, my_ref, ex_ref, ey_ref, sx_ref, sy_ref,
               loc_ref, idx_ref, w_ref):
    x = x_ref[...]
    m = jnp.mean(x, -1, keepdims=True)
    xc = x - m
    v = jnp.mean(xc * xc, -1, keepdims=True)
    xn = xc * lax.rsqrt(v + 1e-5) * g1_ref[...] + bb1_ref[...]
    q = xn + pos_ref[...]
    off256 = jnp.dot(q, woff_ref[...], preferred_element_type=jnp.float32) + boff_ref[...]
    off_hi = off256.astype(jnp.bfloat16).astype(jnp.float32)
    off_lo = off256 - off_hi
    offx = (jnp.dot(off_hi, sx_ref[...], preferred_element_type=jnp.float32)
            + jnp.dot(off_lo, sx_ref[...], preferred_element_type=jnp.float32))
    offy = (jnp.dot(off_hi, sy_ref[...], preferred_element_type=jnp.float32)
            + jnp.dot(off_lo, sy_ref[...], preferred_element_type=jnp.float32))
    logit = jnp.dot(q, wa_ref[...], preferred_element_type=jnp.float32) + ba_ref[...]
    e = jnp.exp(logit)
    s = jnp.dot(e, gsum_ref[...], preferred_element_type=jnp.float32)
    sbc = jnp.dot(s, gbc_ref[...], preferred_element_type=jnp.float32)
    attn = e / sbc
    wi = wi_ref[...]; hi = hi_ref[...]
    wf = wi.astype(jnp.float32); hf = hi.astype(jnp.float32)
    ref8 = ref8_ref[...]
    refx = jnp.dot(ref8, mx_ref[...], preferred_element_type=jnp.float32,
                   precision=lax.Precision.HIGHEST)
    refy = jnp.dot(ref8, my_ref[...], preferred_element_type=jnp.float32,
                   precision=lax.Precision.HIGHEST)
    locx = refx + offx * (1.0 / wf)
    locy = refy + offy * (1.0 / hf)
    ix = locx * wf - 0.5
    iy = locy * hf - 0.5
    x0f = jnp.floor(ix); y0f = jnp.floor(iy)
    fx1 = ix - x0f; fx0 = 1.0 - fx1
    fy1 = iy - y0f; fy0 = 1.0 - fy1
    x0 = x0f.astype(jnp.int32); y0 = y0f.astype(jnp.int32)
    base = base_ref[...] + bcol_ref[...]
    for ci, (dy, dx) in enumerate(((0, 0), (0, 1), (1, 0), (1, 1))):
        xx = x0 + dx; yy = y0 + dy
        valid = (xx >= 0) & (xx < wi) & (yy >= 0) & (yy < hi)
        xcl = jnp.clip(xx, 0, wi - 1); ycl = jnp.clip(yy, 0, hi - 1)
        rowi = base + (ycl * wi + xcl) * _NH
        w = attn * ((fx1 if dx else fx0) * (fy1 if dy else fy0))
        idx_ref[ci] = rowi
        w_ref[ci] = jnp.where(valid, w, 0.0)
    lx_hi = locx.astype(jnp.bfloat16).astype(jnp.float32)
    lx_lo = locx - lx_hi
    ly_hi = locy.astype(jnp.bfloat16).astype(jnp.float32)
    ly_lo = locy - ly_hi
    loc_ref[...] = (jnp.dot(lx_hi, ex_ref[...], preferred_element_type=jnp.float32)
                    + jnp.dot(lx_lo, ex_ref[...], preferred_element_type=jnp.float32)
                    + jnp.dot(ly_hi, ey_ref[...], preferred_element_type=jnp.float32)
                    + jnp.dot(ly_lo, ey_ref[...], preferred_element_type=jnp.float32))


def _prep_call(x2, pos2, ref8, bcol, g1, b1, woff, boff, wa, ba):
    br = 720
    grid = (_R // br,)
    row = lambda i: (i, 0)
    full = lambda i: (0, 0)
    return pl.pallas_call(
        _prep_body,
        grid=grid,
        in_specs=[
            pl.BlockSpec((br, _DIM), row), pl.BlockSpec((br, _DIM), row),
            pl.BlockSpec((br, 8), row),
            pl.BlockSpec((br, 1), row),
            pl.BlockSpec((1, _DIM), full), pl.BlockSpec((1, _DIM), full),
            pl.BlockSpec((_DIM, 256), full), pl.BlockSpec((1, 256), full),
            pl.BlockSpec((_DIM, 128), full), pl.BlockSpec((1, 128), full),
            pl.BlockSpec((1, 128), full), pl.BlockSpec((1, 128), full),
            pl.BlockSpec((1, 128), full),
            pl.BlockSpec((128, 8), full), pl.BlockSpec((8, 128), full),
            pl.BlockSpec((8, 128), full), pl.BlockSpec((8, 128), full),
            pl.BlockSpec((128, 256), full), pl.BlockSpec((128, 256), full),
            pl.BlockSpec((256, 128), full), pl.BlockSpec((256, 128), full),
        ],
        out_specs=[
            pl.BlockSpec((br, 256), row),
            pl.BlockSpec((4, br, 128), lambda i: (0, i, 0)),
            pl.BlockSpec((4, br, 128), lambda i: (0, i, 0)),
        ],
        out_shape=[
            jax.ShapeDtypeStruct((_R, 256), jnp.float32),
            jax.ShapeDtypeStruct((4, _R, 128), jnp.int32),
            jax.ShapeDtypeStruct((4, _R, 128), jnp.float32),
        ],
    )(x2, pos2, ref8, bcol, g1, b1, woff, boff, wa, ba,
      jnp.asarray(_Wi_c), jnp.asarray(_Hi_c), jnp.asarray(_base_c),
      jnp.asarray(_gsum_c), jnp.asarray(_gbc_c),
      jnp.asarray(_mx_c), jnp.asarray(_my_c),
      jnp.asarray(_ex_c), jnp.asarray(_ey_c),
      jnp.asarray(_sx_c), jnp.asarray(_sy_c))


def _valproj_body(s_ref, wlo_ref, whi_ref, blo_ref, bhi_ref, o_ref):
    s = s_ref[...]
    lo = jnp.dot(s, wlo_ref[...], preferred_element_type=jnp.float32) + blo_ref[...]
    hi = jnp.dot(s, whi_ref[...], preferred_element_type=jnp.float32) + bhi_ref[...]
    lou = jax.lax.bitcast_convert_type(lo.astype(jnp.bfloat16), jnp.uint16)
    hiu = jax.lax.bitcast_convert_type(hi.astype(jnp.bfloat16), jnp.uint16)
    o_ref[...] = lou.astype(jnp.int32) | (hiu.astype(jnp.int32) << 16)


def _valproj_call(src2, wlo, whi, blo, bhi, blk0):
    br = 512
    nout = src2.shape[0] // 2
    grid = (nout // br,)
    return pl.pallas_call(
        _valproj_body,
        grid=grid,
        in_specs=[
            pl.BlockSpec((br, _DIM), lambda i: (i + blk0, 0)),
            pl.BlockSpec((_DIM, 128), lambda i: (0, 0)),
            pl.BlockSpec((_DIM, 128), lambda i: (0, 0)),
            pl.BlockSpec((1, 128), lambda i: (0, 0)),
            pl.BlockSpec((1, 128), lambda i: (0, 0)),
        ],
        out_specs=pl.BlockSpec((br, 128), lambda i: (i, 0)),
        out_shape=jax.ShapeDtypeStruct((nout, 128), jnp.int32),
    )(src2, wlo, whi, blo, bhi)


def _sc_gather_body(value_hbm, idx_hbm, w_hbm, out_hbm,
                    idx_v0, w_v0, idx_v1, w_v1, rows0, rows1, out_v, sem0, sem1,
                    q0=0):
    wid = lax.axis_index("s") * 2 + lax.axis_index("c")

    @pl.when(wid < _NAW)
    def _worker():
        lbase = wid * (_NCH * 2)
        qbase = q0 + lbase

        def stage(g, iv, wv):
            r0 = qbase + g * (_GC * 2)
            pltpu.sync_copy(idx_hbm.at[:, pl.ds(r0, _GC * 2), :], iv)
            pltpu.sync_copy(w_hbm.at[:, pl.ds(r0, _GC * 2), :], wv)

        def fire(cw, rows_v, sem, iv):
            for ci in range(4):
                for kb in range(2):
                    pltpu.async_copy(
                        value_hbm.at[iv.at[ci, cw * 2 + kb]],
                        rows_v.at[pl.ds((ci * 2 + kb) * _BS, _BS)], sem)

        def drain(rows_v, sem):
            pltpu.make_async_copy(value_hbm.at[pl.ds(0, _CR * 64)], rows_v, sem).wait()

        def compute(cw, rows_v, wv):
            for rh in range(2):
                def row_body(r2, carry, rh=rh):
                    a0 = jnp.zeros((16,), jnp.float32)
                    a1 = jnp.zeros((16,), jnp.float32)
                    lr = rh * 8 + r2
                    for ci in range(4):
                        ww = wv[ci, cw * 2 + rh, pl.ds(r2 * 16, 16)]
                        for j in range(16):
                            c = ci * 256 + lr * 16 + j
                            w = ww[j]
                            xw = rows_v[c, pl.ds(0, 16)]
                            va = plsc.bitcast(xw << 16, jnp.float32)
                            vb = plsc.bitcast(xw & jnp.int32(-65536), jnp.float32)
                            a0 = a0 + va * w
                            a1 = a1 + vb * w
                    out_v[cw * 2 + rh, pl.ds(r2 * 32, 16)] = a0
                    out_v[cw * 2 + rh, pl.ds(r2 * 32 + 16, 16)] = a1
                    return carry
                lax.fori_loop(0, 8, row_body, 0)

        bufs = ((idx_v0, w_v0), (idx_v1, w_v1))
        stage(0, idx_v0, w_v0)
        fire(0, rows0, sem0, idx_v0)
        for g in range(_NGRP):
            iv, wv = bufs[g % 2]
            ivn, wvn = bufs[(g + 1) % 2]
            if g < _NGRP - 1:
                stage(g + 1, ivn, wvn)

            def pair_body(i, carry, iv=iv, wv=wv, ivn=ivn, g=g):
                cw_a = 2 * i
                fire(cw_a + 1, rows1, sem1, iv)
                drain(rows0, sem0)
                compute(cw_a, rows0, wv)

                @pl.when(i < _GC // 2 - 1)
                def _():
                    fire(cw_a + 2, rows0, sem0, iv)
                if g < _NGRP - 1:
                    @pl.when(i == _GC // 2 - 1)
                    def _():
                        fire(0, rows0, sem0, ivn)
                drain(rows1, sem1)
                compute(cw_a + 1, rows1, wv)
                return carry

            lax.fori_loop(0, _GC // 2, pair_body, 0)
            pltpu.sync_copy(out_v, out_hbm.at[pl.ds(lbase + g * (_GC * 2), _GC * 2)])


def _sc_gather_call(value2, idxc, wc, q0):
    mesh = plsc.VectorSubcoreMesh(core_axis_name="c", subcore_axis_name="s")
    f = functools.partial(
        pl.kernel,
        mesh=mesh,
        compiler_params=pltpu.CompilerParams(use_tc_tiling_on_sc=False,
                                             needs_layout_passes=False),
        out_type=jax.ShapeDtypeStruct((_R // 2, _DIM), jnp.float32),
        scratch_types=[
            pltpu.VMEM((4, _GC * 2, _BS), jnp.int32),
            pltpu.VMEM((4, _GC * 2, _BS), jnp.float32),
            pltpu.VMEM((4, _GC * 2, _BS), jnp.int32),
            pltpu.VMEM((4, _GC * 2, _BS), jnp.float32),
            pltpu.VMEM((_CR * 64, _DH // 2), jnp.int32),
            pltpu.VMEM((_CR * 64, _DH // 2), jnp.int32),
            pltpu.VMEM((_GC * 2, _DIM), jnp.float32),
            pltpu.SemaphoreType.DMA,
            pltpu.SemaphoreType.DMA,
        ],
    )(functools.partial(_sc_gather_body, q0=q0))
    return f(value2, idxc, wc)


def _post_body(x_ref, samp_ref, wo_ref, bo_ref, g2_ref, bb2_ref,
               w1_ref, b1_ref, w2_ref, b2_ref, o_ref):
    x = x_ref[...] + jnp.dot(samp_ref[...], wo_ref[...],
                             preferred_element_type=jnp.float32) + bo_ref[...]
    m = jnp.mean(x, -1, keepdims=True)
    xc = x - m
    v = jnp.mean(xc * xc, -1, keepdims=True)
    xn = xc * lax.rsqrt(v + 1e-5) * g2_ref[...] + bb2_ref[...]
    h1 = jnp.maximum(jnp.dot(xn, w1_ref[...],
                             preferred_element_type=jnp.float32) + b1_ref[...], 0.0)
    o_ref[...] = x + jnp.dot(h1, w2_ref[...],
                             preferred_element_type=jnp.float32) + b2_ref[...]


def _post_call(x2, samp2, wo, bo, g2, b2g, w1, b1, w2, b2, blk0):
    br = 360
    grid = (_R // 2 // br,)
    full = lambda i: (0, 0)
    return pl.pallas_call(
        _post_body,
        grid=grid,
        in_specs=[
            pl.BlockSpec((br, _DIM), lambda i: (i + blk0, 0)),
            pl.BlockSpec((br, _DIM), lambda i: (i, 0)),
            pl.BlockSpec((_DIM, _DIM), full), pl.BlockSpec((1, _DIM), full),
            pl.BlockSpec((1, _DIM), full), pl.BlockSpec((1, _DIM), full),
            pl.BlockSpec((_DIM, _DFF), full), pl.BlockSpec((1, _DFF), full),
            pl.BlockSpec((_DFF, _DIM), full), pl.BlockSpec((1, _DIM), full),
        ],
        out_specs=pl.BlockSpec((br, _DIM), lambda i: (i, 0)),
        out_shape=jax.ShapeDtypeStruct((_R // 2, _DIM), jnp.float32),
    )(x2, samp2, wo, bo, g2, b2g, w1, b1, w2, b2)


def kernel(input, pos, reference_point, source, source_shape, level_start, source_mask,
           ln1_g, ln1_b, W_off, b_off, W_attn, b_attn, W_val, b_val, W_out, b_out,
           ln2_g, ln2_b, W1, b1, W2, b2):
    x2 = input.reshape(_R, _DIM)
    pos2 = pos.reshape(_R, _DIM)
    ref8 = reference_point.reshape(_R, _NL * 2)
    bcol = jnp.asarray((((np.arange(_R, dtype=np.int32)[:, None] // _LQ) % 2)
                        * (_LIN * _NH)))

    loc256, idxc, wc = _prep_call(
        x2, pos2, ref8, bcol, ln1_g[None], ln1_b[None],
        W_off, b_off[None], W_attn, b_attn[None])

    collo = jnp.asarray(_collo_c); colhi = jnp.asarray(_colhi_c)
    src2 = source.reshape(_B * _LIN, _DIM)
    wlo = W_val[:, collo]; whi = W_val[:, colhi]
    blo = b_val[collo][None]; bhi = b_val[colhi][None]
    nvh = _B * _LIN // 2 * _NH

    val_a = _valproj_call(src2, wlo, whi, blo, bhi, 0)
    samp_a = _sc_gather_call(val_a.reshape(nvh, _DH // 2), idxc, wc, 0)
    val_b = _valproj_call(src2, wlo, whi, blo, bhi, _B * _LIN // 2 // 512)
    samp_b = _sc_gather_call(val_b.reshape(nvh, _DH // 2), idxc, wc, _R // 2)

    pargs = (W_out, b_out[None], ln2_g[None], ln2_b[None],
             W1, b1[None], W2, b2[None])
    out_a = _post_call(x2, samp_a, *pargs, 0)
    out_b = _post_call(x2, samp_b, *pargs, _R // 2 // 360)

    loc = loc256.reshape(_B, _LQ, _NH, _NL, _NP, 2)
    outf = jnp.concatenate([out_a, out_b], axis=0)
    return outf.reshape(_B, _LQ, _DIM), loc


# speedup vs baseline: 1.0848x; 1.0848x over previous
import functools

import numpy as np
import jax
import jax.numpy as jnp
from jax import lax
from jax.experimental import pallas as pl
from jax.experimental.pallas import tpu as pltpu
from jax.experimental.pallas import tpu_sc as plsc

_DIM = 256; _DFF = 2048; _NL = 4; _NH = 8; _NP = 4; _DH = 32
_B = 4; _LQ = 900; _R = _B * _LQ
_SHAPES = np.array([[128, 128], [64, 64], [32, 32], [16, 16]], np.int64)
_LIN = int((_SHAPES[:, 0] * _SHAPES[:, 1]).sum())
_LSTART = np.concatenate([[0], np.cumsum(_SHAPES[:, 0] * _SHAPES[:, 1])[:-1]]).astype(np.int64)
_NROWS = _B * _LIN * _NH
_LP = _NL * _NP

_l_lane = np.tile(np.repeat(np.arange(_NL), _NP), _NH)
_h_lane = np.repeat(np.arange(_NH), _LP)
_Wi_c = _SHAPES[_l_lane, 1].astype(np.int32)[None, :]
_Hi_c = _SHAPES[_l_lane, 0].astype(np.int32)[None, :]
_base_c = (_LSTART[_l_lane] * _NH + _h_lane).astype(np.int32)[None, :]
_gsum_c = np.zeros((128, 8), np.float32)
_gsum_c[np.arange(128), _h_lane] = 1.0
_gbc_c = _gsum_c.T.copy()
_mx_c = np.zeros((8, 128), np.float32)
_mx_c[_l_lane * 2, np.arange(128)] = 1.0
_my_c = np.zeros((8, 128), np.float32)
_my_c[_l_lane * 2 + 1, np.arange(128)] = 1.0
_ex_c = np.zeros((128, 256), np.float32)
_ex_c[np.arange(128), 2 * np.arange(128)] = 1.0
_ey_c = np.zeros((128, 256), np.float32)
_ey_c[np.arange(128), 2 * np.arange(128) + 1] = 1.0
_sx_c = np.zeros((256, 128), np.float32)
_sx_c[2 * np.arange(128), np.arange(128)] = 1.0
_sy_c = np.zeros((256, 128), np.float32)
_sy_c[2 * np.arange(128) + 1, np.arange(128)] = 1.0
_collo_c = np.empty((128,), np.int32)
_colhi_c = np.empty((128,), np.int32)
for _h in range(_NH):
    for _k in range(16):
        _collo_c[_h * 16 + _k] = _h * 32 + _k
        _colhi_c[_h * 16 + _k] = _h * 32 + 16 + _k

_NAW = 30; _CR = 16; _GC = 6; _NGRP = 5; _NCH = _GC * _NGRP
_BS = 128


def _prep_body(x_ref, pos_ref, ref8_ref, bcol_ref, g1_ref, bb1_ref,
               woff_ref, boff_ref, wa_ref, ba_ref,
               wi_ref, hi_ref, base_ref, gsum_ref, gbc_ref,
               mx_ref, my_ref, ex_ref, ey_ref, sx_ref, sy_ref,
               loc_ref, idx_ref, w_ref):
    x = x_ref[...]
    m = jnp.mean(x, -1, keepdims=True)
    xc = x - m
    v = jnp.mean(xc * xc, -1, keepdims=True)
    xn = xc * lax.rsqrt(v + 1e-5) * g1_ref[...] + bb1_ref[...]
    q = xn + pos_ref[...]
    off256 = jnp.dot(q, woff_ref[...], preferred_element_type=jnp.float32) + boff_ref[...]
    off_hi = off256.astype(jnp.bfloat16).astype(jnp.float32)
    off_lo = off256 - off_hi
    offx = (jnp.dot(off_hi, sx_ref[...], preferred_element_type=jnp.float32)
            + jnp.dot(off_lo, sx_ref[...], preferred_element_type=jnp.float32))
    offy = (jnp.dot(off_hi, sy_ref[...], preferred_element_type=jnp.float32)
            + jnp.dot(off_lo, sy_ref[...], preferred_element_type=jnp.float32))
    logit = jnp.dot(q, wa_ref[...], preferred_element_type=jnp.float32) + ba_ref[...]
    e = jnp.exp(logit)
    s = jnp.dot(e, gsum_ref[...], preferred_element_type=jnp.float32)
    sbc = jnp.dot(s, gbc_ref[...], preferred_element_type=jnp.float32)
    attn = e / sbc
    wi = wi_ref[...]; hi = hi_ref[...]
    wf = wi.astype(jnp.float32); hf = hi.astype(jnp.float32)
    ref8 = ref8_ref[...]
    refx = jnp.dot(ref8, mx_ref[...], preferred_element_type=jnp.float32,
                   precision=lax.Precision.HIGHEST)
    refy = jnp.dot(ref8, my_ref[...], preferred_element_type=jnp.float32,
                   precision=lax.Precision.HIGHEST)
    locx = refx + offx * (1.0 / wf)
    locy = refy + offy * (1.0 / hf)
    ix = locx * wf - 0.5
    iy = locy * hf - 0.5
    x0f = jnp.floor(ix); y0f = jnp.floor(iy)
    fx1 = ix - x0f; fx0 = 1.0 - fx1
    fy1 = iy - y0f; fy0 = 1.0 - fy1
    x0 = x0f.astype(jnp.int32); y0 = y0f.astype(jnp.int32)
    base = base_ref[...] + bcol_ref[...]
    for ci, (dy, dx) in enumerate(((0, 0), (0, 1), (1, 0), (1, 1))):
        xx = x0 + dx; yy = y0 + dy
        valid = (xx >= 0) & (xx < wi) & (yy >= 0) & (yy < hi)
        xcl = jnp.clip(xx, 0, wi - 1); ycl = jnp.clip(yy, 0, hi - 1)
        rowi = base + (ycl * wi + xcl) * _NH
        w = attn * ((fx1 if dx else fx0) * (fy1 if dy else fy0))
        idx_ref[ci] = rowi
        w_ref[ci] = jnp.where(valid, w, 0.0)
    lx_hi = locx.astype(jnp.bfloat16).astype(jnp.float32)
    lx_lo = locx - lx_hi
    ly_hi = locy.astype(jnp.bfloat16).astype(jnp.float32)
    ly_lo = locy - ly_hi
    loc_ref[...] = (jnp.dot(lx_hi, ex_ref[...], preferred_element_type=jnp.float32)
                    + jnp.dot(lx_lo, ex_ref[...], preferred_element_type=jnp.float32)
                    + jnp.dot(ly_hi, ey_ref[...], preferred_element_type=jnp.float32)
                    + jnp.dot(ly_lo, ey_ref[...], preferred_element_type=jnp.float32))


def _prep_call(x2, pos2, ref8, bcol, g1, b1, woff, boff, wa, ba):
    br = 720
    grid = (_R // br,)
    row = lambda i: (i, 0)
    full = lambda i: (0, 0)
    return pl.pallas_call(
        _prep_body,
        grid=grid,
        in_specs=[
            pl.BlockSpec((br, _DIM), row), pl.BlockSpec((br, _DIM), row),
            pl.BlockSpec((br, 8), row),
            pl.BlockSpec((br, 1), row),
            pl.BlockSpec((1, _DIM), full), pl.BlockSpec((1, _DIM), full),
            pl.BlockSpec((_DIM, 256), full), pl.BlockSpec((1, 256), full),
            pl.BlockSpec((_DIM, 128), full), pl.BlockSpec((1, 128), full),
            pl.BlockSpec((1, 128), full), pl.BlockSpec((1, 128), full),
            pl.BlockSpec((1, 128), full),
            pl.BlockSpec((128, 8), full), pl.BlockSpec((8, 128), full),
            pl.BlockSpec((8, 128), full), pl.BlockSpec((8, 128), full),
            pl.BlockSpec((128, 256), full), pl.BlockSpec((128, 256), full),
            pl.BlockSpec((256, 128), full), pl.BlockSpec((256, 128), full),
        ],
        out_specs=[
            pl.BlockSpec((br, 256), row),
            pl.BlockSpec((4, br, 128), lambda i: (0, i, 0)),
            pl.BlockSpec((4, br, 128), lambda i: (0, i, 0)),
        ],
        out_shape=[
            jax.ShapeDtypeStruct((_R, 256), jnp.float32),
            jax.ShapeDtypeStruct((4, _R, 128), jnp.int32),
            jax.ShapeDtypeStruct((4, _R, 128), jnp.float32),
        ],
    )(x2, pos2, ref8, bcol, g1, b1, woff, boff, wa, ba,
      jnp.asarray(_Wi_c), jnp.asarray(_Hi_c), jnp.asarray(_base_c),
      jnp.asarray(_gsum_c), jnp.asarray(_gbc_c),
      jnp.asarray(_mx_c), jnp.asarray(_my_c),
      jnp.asarray(_ex_c), jnp.asarray(_ey_c),
      jnp.asarray(_sx_c), jnp.asarray(_sy_c))


def _valproj_body(s_ref, wlo_ref, whi_ref, blo_ref, bhi_ref, o_ref):
    s = s_ref[...]
    lo = jnp.dot(s, wlo_ref[...], preferred_element_type=jnp.float32) + blo_ref[...]
    hi = jnp.dot(s, whi_ref[...], preferred_element_type=jnp.float32) + bhi_ref[...]
    lou = jax.lax.bitcast_convert_type(lo.astype(jnp.bfloat16), jnp.uint16)
    hiu = jax.lax.bitcast_convert_type(hi.astype(jnp.bfloat16), jnp.uint16)
    o_ref[...] = lou.astype(jnp.int32) | (hiu.astype(jnp.int32) << 16)


def _valproj_call(src2, wlo, whi, blo, bhi, half):
    br = 1280
    nout = src2.shape[0] // 2
    grid = (nout // br,)
    blk0 = half * (nout // br)
    return pl.pallas_call(
        _valproj_body,
        grid=grid,
        in_specs=[
            pl.BlockSpec((br, _DIM), lambda i: (i + blk0, 0)),
            pl.BlockSpec((_DIM, 128), lambda i: (0, 0)),
            pl.BlockSpec((_DIM, 128), lambda i: (0, 0)),
            pl.BlockSpec((1, 128), lambda i: (0, 0)),
            pl.BlockSpec((1, 128), lambda i: (0, 0)),
        ],
        out_specs=pl.BlockSpec((br, 128), lambda i: (i, 0)),
        out_shape=jax.ShapeDtypeStruct((nout, 128), jnp.int32),
    )(src2, wlo, whi, blo, bhi)


def _sc_gather_body(value_hbm, idx_hbm, w_hbm, out_hbm,
                    idx_v0, w_v0, idx_v1, w_v1, rows0, rows1, out_v, sem0, sem1,
                    q0=0):
    wid = lax.axis_index("s") * 2 + lax.axis_index("c")

    @pl.when(wid < _NAW)
    def _worker():
        lbase = wid * (_NCH * 2)
        qbase = q0 + lbase

        def stage(g, iv, wv):
            r0 = qbase + g * (_GC * 2)
            pltpu.sync_copy(idx_hbm.at[:, pl.ds(r0, _GC * 2), :], iv)
            pltpu.sync_copy(w_hbm.at[:, pl.ds(r0, _GC * 2), :], wv)

        def fire(cw, rows_v, sem, iv):
            for ci in range(4):
                for kb in range(2):
                    pltpu.async_copy(
                        value_hbm.at[iv.at[ci, cw * 2 + kb]],
                        rows_v.at[pl.ds((ci * 2 + kb) * _BS, _BS)], sem)

        def drain(rows_v, sem):
            pltpu.make_async_copy(value_hbm.at[pl.ds(0, _CR * 64)], rows_v, sem).wait()

        def compute(cw, rows_v, wv):
            for rh in range(2):
                def row_body(r2, carry, rh=rh):
                    a0 = jnp.zeros((16,), jnp.float32)
                    a1 = jnp.zeros((16,), jnp.float32)
                    lr = rh * 8 + r2
                    for ci in range(4):
                        ww = wv[ci, cw * 2 + rh, pl.ds(r2 * 16, 16)]
                        for j in range(16):
                            c = ci * 256 + lr * 16 + j
                            w = ww[j]
                            xw = rows_v[c, pl.ds(0, 16)]
                            va = plsc.bitcast(xw << 16, jnp.float32)
                            vb = plsc.bitcast(xw & jnp.int32(-65536), jnp.float32)
                            a0 = a0 + va * w
                            a1 = a1 + vb * w
                    out_v[cw * 2 + rh, pl.ds(r2 * 32, 16)] = a0
                    out_v[cw * 2 + rh, pl.ds(r2 * 32 + 16, 16)] = a1
                    return carry
                lax.fori_loop(0, 8, row_body, 0)

        bufs = ((idx_v0, w_v0), (idx_v1, w_v1))
        stage(0, idx_v0, w_v0)
        fire(0, rows0, sem0, idx_v0)
        for g in range(_NGRP):
            iv, wv = bufs[g % 2]
            ivn, wvn = bufs[(g + 1) % 2]
            if g < _NGRP - 1:
                stage(g + 1, ivn, wvn)

            def pair_body(i, carry, iv=iv, wv=wv, ivn=ivn, g=g):
                cw_a = 2 * i
                fire(cw_a + 1, rows1, sem1, iv)
                drain(rows0, sem0)
                compute(cw_a, rows0, wv)

                @pl.when(i < _GC // 2 - 1)
                def _():
                    fire(cw_a + 2, rows0, sem0, iv)
                if g < _NGRP - 1:
                    @pl.when(i == _GC // 2 - 1)
                    def _():
                        fire(0, rows0, sem0, ivn)
                drain(rows1, sem1)
                compute(cw_a + 1, rows1, wv)
                return carry

            lax.fori_loop(0, _GC // 2, pair_body, 0)
            pltpu.sync_copy(out_v, out_hbm.at[pl.ds(lbase + g * (_GC * 2), _GC * 2)])


def _sc_gather_call(value2, idxc, wc, q0):
    mesh = plsc.VectorSubcoreMesh(core_axis_name="c", subcore_axis_name="s")
    f = functools.partial(
        pl.kernel,
        mesh=mesh,
        compiler_params=pltpu.CompilerParams(use_tc_tiling_on_sc=False,
                                             needs_layout_passes=False),
        out_type=jax.ShapeDtypeStruct((_R // 2, _DIM), jnp.float32),
        scratch_types=[
            pltpu.VMEM((4, _GC * 2, _BS), jnp.int32),
            pltpu.VMEM((4, _GC * 2, _BS), jnp.float32),
            pltpu.VMEM((4, _GC * 2, _BS), jnp.int32),
            pltpu.VMEM((4, _GC * 2, _BS), jnp.float32),
            pltpu.VMEM((_CR * 64, _DH // 2), jnp.int32),
            pltpu.VMEM((_CR * 64, _DH // 2), jnp.int32),
            pltpu.VMEM((_GC * 2, _DIM), jnp.float32),
            pltpu.SemaphoreType.DMA,
            pltpu.SemaphoreType.DMA,
        ],
    )(functools.partial(_sc_gather_body, q0=q0))
    return f(value2, idxc, wc)


def _post_body(x_ref, samp_ref, wo_ref, bo_ref, g2_ref, bb2_ref,
               w1_ref, b1_ref, w2_ref, b2_ref, o_ref):
    x = x_ref[...] + jnp.dot(samp_ref[...], wo_ref[...],
                             preferred_element_type=jnp.float32) + bo_ref[...]
    m = jnp.mean(x, -1, keepdims=True)
    xc = x - m
    v = jnp.mean(xc * xc, -1, keepdims=True)
    xn = xc * lax.rsqrt(v + 1e-5) * g2_ref[...] + bb2_ref[...]
    h1 = jnp.maximum(jnp.dot(xn, w1_ref[...],
                             preferred_element_type=jnp.float32) + b1_ref[...], 0.0)
    o_ref[...] = x + jnp.dot(h1, w2_ref[...],
                             preferred_element_type=jnp.float32) + b2_ref[...]


def _post_call(x2, samp2, wo, bo, g2, b2g, w1, b1, w2, b2, blk0):
    br = 360
    grid = (_R // 2 // br,)
    full = lambda i: (0, 0)
    return pl.pallas_call(
        _post_body,
        grid=grid,
        in_specs=[
            pl.BlockSpec((br, _DIM), lambda i: (i + blk0, 0)),
            pl.BlockSpec((br, _DIM), lambda i: (i, 0)),
            pl.BlockSpec((_DIM, _DIM), full), pl.BlockSpec((1, _DIM), full),
            pl.BlockSpec((1, _DIM), full), pl.BlockSpec((1, _DIM), full),
            pl.BlockSpec((_DIM, _DFF), full), pl.BlockSpec((1, _DFF), full),
            pl.BlockSpec((_DFF, _DIM), full), pl.BlockSpec((1, _DIM), full),
        ],
        out_specs=pl.BlockSpec((br, _DIM), lambda i: (i, 0)),
        out_shape=jax.ShapeDtypeStruct((_R // 2, _DIM), jnp.float32),
    )(x2, samp2, wo, bo, g2, b2g, w1, b1, w2, b2)


def kernel(input, pos, reference_point, source, source_shape, level_start, source_mask,
           ln1_g, ln1_b, W_off, b_off, W_attn, b_attn, W_val, b_val, W_out, b_out,
           ln2_g, ln2_b, W1, b1, W2, b2):
    x2 = input.reshape(_R, _DIM)
    pos2 = pos.reshape(_R, _DIM)
    ref8 = reference_point.reshape(_R, _NL * 2)
    bcol = jnp.asarray((((np.arange(_R, dtype=np.int32)[:, None] // _LQ) % 2)
                        * (_LIN * _NH)))

    loc256, idxc, wc = _prep_call(
        x2, pos2, ref8, bcol, ln1_g[None], ln1_b[None],
        W_off, b_off[None], W_attn, b_attn[None])

    collo = jnp.asarray(_collo_c); colhi = jnp.asarray(_colhi_c)
    src2 = source.reshape(_B * _LIN, _DIM)
    wlo = W_val[:, collo]; whi = W_val[:, colhi]
    blo = b_val[collo][None]; bhi = b_val[colhi][None]
    nvh = _B * _LIN // 2 * _NH

    val_a = _valproj_call(src2, wlo, whi, blo, bhi, 0)
    samp_a = _sc_gather_call(val_a.reshape(nvh, _DH // 2), idxc, wc, 0)
    val_b = _valproj_call(src2, wlo, whi, blo, bhi, 1)
    samp_b = _sc_gather_call(val_b.reshape(nvh, _DH // 2), idxc, wc, _R // 2)

    pargs = (W_out, b_out[None], ln2_g[None], ln2_b[None],
             W1, b1[None], W2, b2[None])
    out_a = _post_call(x2, samp_a, *pargs, 0)
    out_b = _post_call(x2, samp_b, *pargs, _R // 2 // 360)

    loc = loc256.reshape(_B, _LQ, _NH, _NL, _NP, 2)
    outf = jnp.concatenate([out_a, out_b], axis=0)
    return outf.reshape(_B, _LQ, _DIM), loc
